# baseline (speedup 1.0000x reference)
import functools

import jax
import jax.numpy as jnp
from jax import lax
from jax.experimental import pallas as pl
from jax.experimental.pallas import tpu as pltpu
from jax.experimental.pallas import tpu_sc as plsc

NUM_EMB = 100000
DIM = 128

NUM_CORES = 2
NUM_SUBCORES = 16
NW = NUM_CORES * NUM_SUBCORES

CHUNK = 128


def _tc_dequant_table(weight, scaler):
    BLKT = 2048

    def body(w_ref, s_ref, o_ref):
        s = s_ref[...].reshape(BLKT, 1)
        o_ref[...] = w_ref[...].astype(jnp.float32) * s

    return pl.pallas_call(
        body,
        grid=(pl.cdiv(NUM_EMB, BLKT),),
        in_specs=[
            pl.BlockSpec((BLKT, DIM), lambda i: (i, 0)),
            pl.BlockSpec((BLKT,), lambda i: (i,)),
        ],
        out_specs=pl.BlockSpec((BLKT, DIM), lambda i: (i, 0)),
        out_shape=jax.ShapeDtypeStruct((NUM_EMB, DIM), jnp.float32),
    )(weight, scaler)


def _sc_gather(table_f32, idx3):
    _, n_chunks, _ = idx3.shape
    B = NW * n_chunks * CHUNK
    b_per_w = n_chunks * CHUNK

    mesh = plsc.VectorSubcoreMesh(
        core_axis_name="c",
        subcore_axis_name="s",
        num_cores=NUM_CORES,
        num_subcores=NUM_SUBCORES,
    )

    @functools.partial(
        pl.kernel,
        out_type=jax.ShapeDtypeStruct((B, DIM), jnp.float32),
        mesh=mesh,
        compiler_params=pltpu.CompilerParams(use_tc_tiling_on_sc=False),
        scratch_types=[
            pltpu.VMEM((n_chunks, CHUNK), jnp.int32),
            pltpu.VMEM((4, CHUNK, DIM), jnp.float32),
            pltpu.SemaphoreType.DMA,
            pltpu.SemaphoreType.DMA,
            pltpu.SemaphoreType.DMA,
            pltpu.SemaphoreType.DMA,
        ],
    )
    def k(tab_hbm, idx_hbm, rows_out, idx_v, rows_v, sem0, sem1, sem2, sem3):
        wid = lax.axis_index("s") * NUM_CORES + lax.axis_index("c")
        base = wid * b_per_w
        sem = (sem0, sem1, sem2, sem3)
        pltpu.sync_copy(idx_hbm.at[wid], idx_v)

        def start(j, slot):
            pltpu.async_copy(
                tab_hbm.at[idx_v.at[j]], rows_v.at[slot], sem[slot])

        def finish(j, slot):
            pltpu.make_async_copy(
                tab_hbm.at[idx_v.at[j]], rows_v.at[slot], sem[slot]).wait()
            off = base + j * CHUNK
            pltpu.sync_copy(rows_v.at[slot], rows_out.at[pl.ds(off, CHUNK)])

        for b in range(4):
            start(b, b)

        main_end = 4 * ((n_chunks - 4) // 4)

        @pl.loop(0, main_end, step=4)
        def _chunk(j):
            for b in range(4):
                finish(j + b, b)
                start(j + 4 + b, b)

        for c in range(main_end, n_chunks):
            finish(c, c % 4)
            if c + 4 < n_chunks:
                start(c + 4, (c + 4) % 4)

    return k(table_f32, idx3)


def kernel(x, weight, weight_scaler):
    B0, S = x.shape
    B = B0 * S
    b_per_w = B // NW
    idx3 = x.astype(jnp.int32).reshape(NW, b_per_w // CHUNK, CHUNK)
    table_f32 = _tc_dequant_table(weight, weight_scaler)
    out = _sc_gather(table_f32, idx3)
    return out.reshape(B0, S, DIM)


# speedup vs baseline: 1.0953x; 1.0953x over previous
import functools

import jax
import jax.numpy as jnp
from jax import lax
from jax.experimental import pallas as pl
from jax.experimental.pallas import tpu as pltpu
from jax.experimental.pallas import tpu_sc as plsc

NUM_EMB = 100000
DIM = 128

NUM_CORES = 2
NUM_SUBCORES = 16
NW = NUM_CORES * NUM_SUBCORES

CHUNK = 128


def _tc_dequant_table(weight, scaler):
    BLKT = 4096

    def body(w_ref, s_ref, o_ref):
        s = s_ref[...].reshape(BLKT, 1)
        o_ref[...] = w_ref[...].astype(jnp.float32) * s

    return pl.pallas_call(
        body,
        grid=(pl.cdiv(NUM_EMB, BLKT),),
        in_specs=[
            pl.BlockSpec((BLKT, DIM), lambda i: (i, 0)),
            pl.BlockSpec((BLKT,), lambda i: (i,)),
        ],
        out_specs=pl.BlockSpec((BLKT, DIM), lambda i: (i, 0)),
        out_shape=jax.ShapeDtypeStruct((NUM_EMB, DIM), jnp.float32),
    )(weight, scaler)


def _sc_gather(table_f32, idx3):
    _, n_chunks, _ = idx3.shape
    B = NW * n_chunks * CHUNK
    b_per_w = n_chunks * CHUNK

    mesh = plsc.VectorSubcoreMesh(
        core_axis_name="c",
        subcore_axis_name="s",
        num_cores=NUM_CORES,
        num_subcores=NUM_SUBCORES,
    )

    @functools.partial(
        pl.kernel,
        out_type=jax.ShapeDtypeStruct((B, DIM), jnp.float32),
        mesh=mesh,
        compiler_params=pltpu.CompilerParams(use_tc_tiling_on_sc=False),
        scratch_types=[
            pltpu.VMEM((n_chunks, CHUNK), jnp.int32),
            pltpu.VMEM((4, CHUNK, DIM), jnp.float32),
            pltpu.SemaphoreType.DMA,
            pltpu.SemaphoreType.DMA,
            pltpu.SemaphoreType.DMA,
            pltpu.SemaphoreType.DMA,
        ],
    )
    def k(tab_hbm, idx_hbm, rows_out, idx_v, rows_v, sem0, sem1, sem2, sem3):
        wid = lax.axis_index("s") * NUM_CORES + lax.axis_index("c")
        base = wid * b_per_w
        sem = (sem0, sem1, sem2, sem3)
        pltpu.sync_copy(idx_hbm.at[wid], idx_v)

        def start(j, slot):
            pltpu.async_copy(
                tab_hbm.at[idx_v.at[j]], rows_v.at[slot], sem[slot])

        def finish(j, slot):
            pltpu.make_async_copy(
                tab_hbm.at[idx_v.at[j]], rows_v.at[slot], sem[slot]).wait()
            off = base + j * CHUNK
            pltpu.sync_copy(rows_v.at[slot], rows_out.at[pl.ds(off, CHUNK)])

        for b in range(4):
            start(b, b)

        main_end = 4 * ((n_chunks - 4) // 4)

        @pl.loop(0, main_end, step=4)
        def _chunk(j):
            for b in range(4):
                finish(j + b, b)
                start(j + 4 + b, b)

        for c in range(main_end, n_chunks):
            finish(c, c % 4)
            if c + 4 < n_chunks:
                start(c + 4, (c + 4) % 4)

    return k(table_f32, idx3)


def kernel(x, weight, weight_scaler):
    B0, S = x.shape
    B = B0 * S
    b_per_w = B // NW
    idx3 = x.astype(jnp.int32).reshape(NW, b_per_w // CHUNK, CHUNK)
    table_f32 = _tc_dequant_table(weight, weight_scaler)
    out = _sc_gather(table_f32, idx3)
    return out.reshape(B0, S, DIM)


# speedup vs baseline: 1.1553x; 1.0548x over previous
import functools

import jax
import jax.numpy as jnp
from jax import lax
from jax.experimental import pallas as pl
from jax.experimental.pallas import tpu as pltpu
from jax.experimental.pallas import tpu_sc as plsc

NUM_EMB = 100000
DIM = 128

NUM_CORES = 2
NUM_SUBCORES = 16
NW = NUM_CORES * NUM_SUBCORES

CHUNK = 128


def _tc_dequant_table(weight, scaler):
    BLKT = 8192

    def body(w_ref, s_ref, o_ref):
        s = s_ref[...].reshape(BLKT, 1)
        o_ref[...] = w_ref[...].astype(jnp.float32) * s

    return pl.pallas_call(
        body,
        grid=(pl.cdiv(NUM_EMB, BLKT),),
        in_specs=[
            pl.BlockSpec((BLKT, DIM), lambda i: (i, 0)),
            pl.BlockSpec((BLKT,), lambda i: (i,)),
        ],
        out_specs=pl.BlockSpec((BLKT, DIM), lambda i: (i, 0)),
        out_shape=jax.ShapeDtypeStruct((NUM_EMB, DIM), jnp.float32),
    )(weight, scaler)


def _sc_gather(table_f32, idx3):
    _, n_chunks, _ = idx3.shape
    B = NW * n_chunks * CHUNK
    b_per_w = n_chunks * CHUNK

    mesh = plsc.VectorSubcoreMesh(
        core_axis_name="c",
        subcore_axis_name="s",
        num_cores=NUM_CORES,
        num_subcores=NUM_SUBCORES,
    )

    @functools.partial(
        pl.kernel,
        out_type=jax.ShapeDtypeStruct((B, DIM), jnp.float32),
        mesh=mesh,
        compiler_params=pltpu.CompilerParams(use_tc_tiling_on_sc=False),
        scratch_types=[
            pltpu.VMEM((n_chunks, CHUNK), jnp.int32),
            pltpu.VMEM((4, CHUNK, DIM), jnp.float32),
            pltpu.SemaphoreType.DMA,
            pltpu.SemaphoreType.DMA,
            pltpu.SemaphoreType.DMA,
            pltpu.SemaphoreType.DMA,
        ],
    )
    def k(tab_hbm, idx_hbm, rows_out, idx_v, rows_v, sem0, sem1, sem2, sem3):
        wid = lax.axis_index("s") * NUM_CORES + lax.axis_index("c")
        base = wid * b_per_w
        sem = (sem0, sem1, sem2, sem3)
        pltpu.sync_copy(idx_hbm.at[wid], idx_v)

        def start(j, slot):
            pltpu.async_copy(
                tab_hbm.at[idx_v.at[j]], rows_v.at[slot], sem[slot])

        def finish(j, slot):
            pltpu.make_async_copy(
                tab_hbm.at[idx_v.at[j]], rows_v.at[slot], sem[slot]).wait()
            off = base + j * CHUNK
            pltpu.sync_copy(rows_v.at[slot], rows_out.at[pl.ds(off, CHUNK)])

        for b in range(4):
            start(b, b)

        main_end = 4 * ((n_chunks - 4) // 4)

        @pl.loop(0, main_end, step=4)
        def _chunk(j):
            for b in range(4):
                finish(j + b, b)
                start(j + 4 + b, b)

        for c in range(main_end, n_chunks):
            finish(c, c % 4)
            if c + 4 < n_chunks:
                start(c + 4, (c + 4) % 4)

    return k(table_f32, idx3)


def kernel(x, weight, weight_scaler):
    B0, S = x.shape
    B = B0 * S
    b_per_w = B // NW
    idx3 = x.astype(jnp.int32).reshape(NW, b_per_w // CHUNK, CHUNK)
    table_f32 = _tc_dequant_table(weight, weight_scaler)
    out = _sc_gather(table_f32, idx3)
    return out.reshape(B0, S, DIM)


# speedup vs baseline: 1.1575x; 1.0018x over previous
import functools

import jax
import jax.numpy as jnp
from jax import lax
from jax.experimental import pallas as pl
from jax.experimental.pallas import tpu as pltpu
from jax.experimental.pallas import tpu_sc as plsc

NUM_EMB = 100000
DIM = 128

NUM_CORES = 2
NUM_SUBCORES = 16
NW = NUM_CORES * NUM_SUBCORES

CHUNK = 128


def _tc_dequant_table(weight, scaler):
    BLKT = 8192

    def body(w_ref, s_ref, o_ref):
        s = s_ref[...].reshape(BLKT, 1)
        o_ref[...] = w_ref[...].astype(jnp.float32) * s

    return pl.pallas_call(
        body,
        grid=(pl.cdiv(NUM_EMB, BLKT),),
        in_specs=[
            pl.BlockSpec((BLKT, DIM), lambda i: (i, 0)),
            pl.BlockSpec((BLKT,), lambda i: (i,)),
        ],
        out_specs=pl.BlockSpec((BLKT, DIM), lambda i: (i, 0)),
        out_shape=jax.ShapeDtypeStruct((NUM_EMB, DIM), jnp.float32),
    )(weight, scaler)


def _sc_gather(table_f32, idx3):
    _, n_chunks, _ = idx3.shape
    B = NW * n_chunks * CHUNK
    b_per_w = n_chunks * CHUNK

    mesh = plsc.VectorSubcoreMesh(
        core_axis_name="c",
        subcore_axis_name="s",
        num_cores=NUM_CORES,
        num_subcores=NUM_SUBCORES,
    )

    @functools.partial(
        pl.kernel,
        out_type=jax.ShapeDtypeStruct((B, DIM), jnp.float32),
        mesh=mesh,
        compiler_params=pltpu.CompilerParams(use_tc_tiling_on_sc=False),
        scratch_types=[
            pltpu.VMEM((n_chunks, CHUNK), jnp.int32),
            pltpu.VMEM((6, CHUNK, DIM), jnp.float32),
        ]
        + [pltpu.SemaphoreType.DMA] * 12,
    )
    def k(tab_hbm, idx_hbm, rows_out, idx_v, rows_v, *sems):
        wid = lax.axis_index("s") * NUM_CORES + lax.axis_index("c")
        base = wid * b_per_w
        gsem = sems[:6]
        osem = sems[6:]
        pltpu.sync_copy(idx_hbm.at[wid], idx_v)

        def start_gather(j, slot):
            pltpu.async_copy(
                tab_hbm.at[idx_v.at[j]], rows_v.at[slot], gsem[slot])

        def wait_gather(j, slot):
            pltpu.make_async_copy(
                tab_hbm.at[idx_v.at[j]], rows_v.at[slot], gsem[slot]).wait()

        def _out_copy(j, slot):
            off = base + j * CHUNK
            return pltpu.make_async_copy(
                rows_v.at[slot], rows_out.at[pl.ds(off, CHUNK)], osem[slot])

        def start_out(j, slot):
            _out_copy(j, slot).start()

        def wait_out(j, slot):
            _out_copy(j, slot).wait()

        for b in range(4):
            start_gather(b, b)
        for b in range(2):
            wait_gather(b, b)
            start_out(b, b)
            start_gather(b + 4, b + 4)

        main_end = 2 + 6 * ((n_chunks - 4 - 2) // 6)

        @pl.loop(2, main_end, step=6)
        def _chunk(j0):
            for b in range(6):
                j = j0 + b
                slot = (2 + b) % 6
                pslot = (slot + 4) % 6
                wait_out(j - 2, pslot)
                start_gather(j + 4, pslot)
                wait_gather(j, slot)
                start_out(j, slot)

        for c in range(main_end, n_chunks):
            s = c % 6
            if c + 4 < n_chunks:
                ps = (c + 4) % 6
                wait_out(c - 2, ps)
                start_gather(c + 4, ps)
            wait_gather(c, s)
            start_out(c, s)

        for c in range(max(0, n_chunks - 6), n_chunks):
            wait_out(c, c % 6)

    return k(table_f32, idx3)


def kernel(x, weight, weight_scaler):
    B0, S = x.shape
    B = B0 * S
    b_per_w = B // NW
    idx3 = x.astype(jnp.int32).reshape(NW, b_per_w // CHUNK, CHUNK)
    table_f32 = _tc_dequant_table(weight, weight_scaler)
    out = _sc_gather(table_f32, idx3)
    return out.reshape(B0, S, DIM)


# speedup vs baseline: 1.1820x; 1.0212x over previous
import functools

import jax
import jax.numpy as jnp
from jax import lax
from jax.experimental import pallas as pl
from jax.experimental.pallas import tpu as pltpu
from jax.experimental.pallas import tpu_sc as plsc

NUM_EMB = 100000
DIM = 128

NUM_CORES = 2
NUM_SUBCORES = 16
NW = NUM_CORES * NUM_SUBCORES

CHUNK = 128


def _tc_dequant_table(weight, scaler):
    BLKT = 16384

    def body(w_ref, s_ref, o_ref):
        s = s_ref[...].reshape(BLKT, 1)
        o_ref[...] = w_ref[...].astype(jnp.float32) * s

    return pl.pallas_call(
        body,
        grid=(pl.cdiv(NUM_EMB, BLKT),),
        in_specs=[
            pl.BlockSpec((BLKT, DIM), lambda i: (i, 0)),
            pl.BlockSpec((BLKT,), lambda i: (i,)),
        ],
        out_specs=pl.BlockSpec((BLKT, DIM), lambda i: (i, 0)),
        out_shape=jax.ShapeDtypeStruct((NUM_EMB, DIM), jnp.float32),
    )(weight, scaler)


def _sc_gather(table_f32, idx3):
    _, n_chunks, _ = idx3.shape
    B = NW * n_chunks * CHUNK
    b_per_w = n_chunks * CHUNK

    mesh = plsc.VectorSubcoreMesh(
        core_axis_name="c",
        subcore_axis_name="s",
        num_cores=NUM_CORES,
        num_subcores=NUM_SUBCORES,
    )

    @functools.partial(
        pl.kernel,
        out_type=jax.ShapeDtypeStruct((B, DIM), jnp.float32),
        mesh=mesh,
        compiler_params=pltpu.CompilerParams(use_tc_tiling_on_sc=False),
        scratch_types=[
            pltpu.VMEM((n_chunks, CHUNK), jnp.int32),
            pltpu.VMEM((6, CHUNK, DIM), jnp.float32),
        ]
        + [pltpu.SemaphoreType.DMA] * 12,
    )
    def k(tab_hbm, idx_hbm, rows_out, idx_v, rows_v, *sems):
        wid = lax.axis_index("s") * NUM_CORES + lax.axis_index("c")
        base = wid * b_per_w
        gsem = sems[:6]
        osem = sems[6:]
        pltpu.sync_copy(idx_hbm.at[wid], idx_v)

        def start_gather(j, slot):
            pltpu.async_copy(
                tab_hbm.at[idx_v.at[j]], rows_v.at[slot], gsem[slot])

        def wait_gather(j, slot):
            pltpu.make_async_copy(
                tab_hbm.at[idx_v.at[j]], rows_v.at[slot], gsem[slot]).wait()

        def _out_copy(j, slot):
            off = base + j * CHUNK
            return pltpu.make_async_copy(
                rows_v.at[slot], rows_out.at[pl.ds(off, CHUNK)], osem[slot])

        def start_out(j, slot):
            _out_copy(j, slot).start()

        def wait_out(j, slot):
            _out_copy(j, slot).wait()

        for b in range(4):
            start_gather(b, b)
        for b in range(2):
            wait_gather(b, b)
            start_out(b, b)
            start_gather(b + 4, b + 4)

        main_end = 2 + 6 * ((n_chunks - 4 - 2) // 6)

        @pl.loop(2, main_end, step=6)
        def _chunk(j0):
            for b in range(6):
                j = j0 + b
                slot = (2 + b) % 6
                pslot = (slot + 4) % 6
                wait_out(j - 2, pslot)
                start_gather(j + 4, pslot)
                wait_gather(j, slot)
                start_out(j, slot)

        for c in range(main_end, n_chunks):
            s = c % 6
            if c + 4 < n_chunks:
                ps = (c + 4) % 6
                wait_out(c - 2, ps)
                start_gather(c + 4, ps)
            wait_gather(c, s)
            start_out(c, s)

        for c in range(max(0, n_chunks - 6), n_chunks):
            wait_out(c, c % 6)

    return k(table_f32, idx3)


def kernel(x, weight, weight_scaler):
    B0, S = x.shape
    B = B0 * S
    b_per_w = B // NW
    idx3 = x.astype(jnp.int32).reshape(NW, b_per_w // CHUNK, CHUNK)
    table_f32 = _tc_dequant_table(weight, weight_scaler)
    out = _sc_gather(table_f32, idx3)
    return out.reshape(B0, S, DIM)
